# baseline (speedup 1.0000x reference)
import functools

import jax
import jax.numpy as jnp
from jax import lax
from jax.experimental import pallas as pl
from jax.experimental.pallas import tpu as pltpu
from jax.experimental.pallas import tpu_sc as plsc

N_PATCH = 8
K_NN = 16
NEWTON_ITERS = 18
INTERPRET = False

_NC = 2
_NS = 16
_NW = _NC * _NS
_RB = 16


def _d2_body(coords_ref, coords_t_ref, out_ref):
    co = coords_ref[0]
    ct = coords_t_ref[0]
    x = co[0:1, :]
    y = co[1:2, :]
    z = co[2:3, :]
    xc = ct[:, 0:1]
    yc = ct[:, 1:2]
    zc = ct[:, 2:3]
    dx = xc - x
    dy = yc - y
    dz = zc - z
    out_ref[0] = dx * dx + dy * dy + dz * dz


def _sort16(a):
    return plsc.sort_key_val(a, a)[0]


def _merge16(a, b):
    return _sort16(jnp.minimum(a, lax.rev(b, (0,))))


def _row_kth(buf, base):
    lvl = [_sort16(buf[pl.ds(base + 16 * c, 16)]) for c in range(64)]
    while len(lvl) > 1:
        lvl = [_merge16(lvl[2 * i], lvl[2 * i + 1]) for i in range(len(lvl) // 2)]
    return jnp.max(lvl[0])


def _sc_select_body(rows_per_w, d2_hbm, t_hbm, buf0, buf1, tv0, tv1, sem0, sem1):
    wid = lax.axis_index("s") * _NC + lax.axis_index("c")
    row0 = wid * rows_per_w
    n_blocks = rows_per_w // _RB
    lane = lax.iota(jnp.int32, 16)

    def dma(g, buf, sem):
        return pltpu.make_async_copy(
            d2_hbm.at[pl.ds((row0 + g * _RB) * 1024, _RB * 1024)], buf, sem
        )

    dma(0, buf0, sem0).start()
    dma(1, buf1, sem1).start()

    def do_block(g, buf, tv, sem):
        dma(g, buf, sem).wait()

        def row_fn(i, tvec):
            t = _row_kth(buf, i * 1024)
            return jnp.where(lane == i, t, tvec)

        tvec = lax.fori_loop(0, _RB, row_fn, jnp.zeros((16,), jnp.float32))

        @pl.when(g + 2 < n_blocks)
        def _():
            dma(g + 2, buf, sem).start()

        tv[...] = tvec
        pltpu.sync_copy(tv, t_hbm.at[pl.ds(row0 + g * _RB, _RB)])

    def super_body(s, carry):
        do_block(s * 2, buf0, tv0, sem0)
        do_block(s * 2 + 1, buf1, tv1, sem1)
        return carry

    lax.fori_loop(0, n_blocks // 2, super_body, jnp.int32(0))


def _cloud_props(co, ct, t):
    x = co[0:1, :]
    y = co[1:2, :]
    z = co[2:3, :]
    xc = ct[:, 0:1]
    yc = ct[:, 1:2]
    zc = ct[:, 2:3]

    dx = xc - x
    dy = yc - y
    dz = zc - z
    d2 = dx * dx + dy * dy + dz * dz

    maskf = jnp.where(d2 <= t, jnp.float32(1.0), jnp.float32(0.0))

    one = jnp.ones_like(x)
    feats = jnp.concatenate(
        [x, y, z, x * x, y * y, z * z, x * y, x * z, y * z, one], axis=0
    )
    S = jnp.dot(feats, maskf, preferred_element_type=jnp.float32)

    Sx = S[0:1]
    Sy = S[1:2]
    Sz = S[2:3]
    cn = S[9:10]
    cxx = S[3:4] - 2.0 * x * Sx + cn * x * x
    cyy = S[4:5] - 2.0 * y * Sy + cn * y * y
    czz = S[5:6] - 2.0 * z * Sz + cn * z * z
    cxy = S[6:7] - x * Sy - y * Sx + cn * x * y
    cxz = S[7:8] - x * Sz - z * Sx + cn * x * z
    cyz = S[8:9] - y * Sz - z * Sy + cn * y * z

    c2 = cxx + cyy + czz
    c1 = (cxx * cyy - cxy * cxy) + (cxx * czz - cxz * cxz) + (cyy * czz - cyz * cyz)
    c0 = (
        cxx * (cyy * czz - cyz * cyz)
        - cxy * (cxy * czz - cyz * cxz)
        + cxz * (cxy * cyz - cyy * cxz)
    )
    lam = -0.01 * c2 - jnp.float32(1e-12)
    for _ in range(NEWTON_ITERS):
        fv = ((c2 - lam) * lam - c1) * lam + c0
        fp = (2.0 * c2 - 3.0 * lam) * lam - c1
        fp = jnp.minimum(fp, jnp.float32(-1e-30))
        lam = lam - fv / fp

    m00 = cxx - lam
    m11 = cyy - lam
    m22 = czz - lam
    a01x = cxy * cyz - cxz * m11
    a01y = cxz * cxy - m00 * cyz
    a01z = m00 * m11 - cxy * cxy
    a02x = cxy * m22 - cxz * cyz
    a02y = cxz * cxz - m00 * m22
    a02z = m00 * cyz - cxy * cxz
    a12x = m11 * m22 - cyz * cyz
    a12y = cyz * cxz - cxy * m22
    a12z = cxy * cyz - m11 * cxz
    n01 = a01x * a01x + a01y * a01y + a01z * a01z
    n02 = a02x * a02x + a02y * a02y + a02z * a02z
    n12 = a12x * a12x + a12y * a12y + a12z * a12z

    use02 = n02 > n01
    vx = jnp.where(use02, a02x, a01x)
    vy = jnp.where(use02, a02y, a01y)
    vz = jnp.where(use02, a02z, a01z)
    nb = jnp.maximum(n01, n02)
    use12 = n12 > nb
    vx = jnp.where(use12, a12x, vx)
    vy = jnp.where(use12, a12y, vy)
    vz = jnp.where(use12, a12z, vz)
    nb = jnp.maximum(nb, n12)

    inv = lax.rsqrt(nb + jnp.float32(1e-38))
    anx = jnp.abs(vx) * inv
    any_ = jnp.abs(vy) * inv
    anz = jnp.abs(vz) * inv
    sv = lam / jnp.maximum(c2, jnp.float32(1e-38))
    return anx, any_, anz, sv


def _pair_body(coords_ref, coords_t_ref, t_ref, out_ref):
    step = pl.program_id(0)
    sx, sy, sz, ssv = _cloud_props(
        coords_ref[0, 0], coords_t_ref[0, 0], t_ref[0, 0]
    )
    dx_, dy_, dz_, dsv = _cloud_props(
        coords_ref[0, 1], coords_t_ref[0, 1], t_ref[0, 1]
    )

    nl = jnp.sqrt((sx - dx_) ** 2 + (sy - dy_) ** 2 + (sz - dz_) ** 2)
    svl = jnp.abs(ssv - dsv)
    zeros = jnp.zeros_like(nl)
    acc = jnp.concatenate([nl, svl, zeros, zeros, zeros, zeros, zeros, zeros], axis=0)

    @pl.when(step == 0)
    def _():
        out_ref[...] = jnp.zeros_like(out_ref)

    out_ref[...] += acc


def kernel(srcPC, dstPC):
    B, N, _ = srcPC.shape
    n_pairs = B * N_PATCH
    n_cp = 2 * n_pairs
    P = N // N_PATCH
    rows_per_w = n_cp * P // _NW

    s = srcPC.reshape(n_pairs, P, 3)
    d = dstPC.reshape(n_pairs, P, 3)
    pts = jnp.stack([s, d], axis=1)
    pts = jnp.pad(pts, ((0, 0), (0, 0), (0, 0), (0, 5)))
    coords_t = pts.reshape(n_cp, P, 8)
    coords = jnp.swapaxes(coords_t, 1, 2)

    d2 = pl.pallas_call(
        _d2_body,
        grid=(n_cp,),
        in_specs=[
            pl.BlockSpec((1, 8, P), lambda i: (i, 0, 0)),
            pl.BlockSpec((1, P, 8), lambda i: (i, 0, 0)),
        ],
        out_specs=pl.BlockSpec((1, P, P), lambda i: (i, 0, 0)),
        out_shape=jax.ShapeDtypeStruct((n_cp, P, P), jnp.float32),
        compiler_params=pltpu.CompilerParams(
            dimension_semantics=("arbitrary",),
        ),
        interpret=INTERPRET,
    )(coords, coords_t)

    mesh = plsc.VectorSubcoreMesh(core_axis_name="c", subcore_axis_name="s")
    sel = functools.partial(
        pl.kernel,
        mesh=mesh,
        out_type=jax.ShapeDtypeStruct((n_cp * P,), jnp.float32),
        compiler_params=pltpu.CompilerParams(needs_layout_passes=False),
        scratch_types=[
            pltpu.VMEM((_RB * 1024,), jnp.float32),
            pltpu.VMEM((_RB * 1024,), jnp.float32),
            pltpu.VMEM((16,), jnp.float32),
            pltpu.VMEM((16,), jnp.float32),
            pltpu.SemaphoreType.DMA,
            pltpu.SemaphoreType.DMA,
        ],
    )(functools.partial(_sc_select_body, rows_per_w))
    t_flat = sel(d2.reshape(n_cp * P * P))

    t = t_flat.reshape(n_pairs, 2, 1, P)

    res = pl.pallas_call(
        _pair_body,
        grid=(n_pairs,),
        in_specs=[
            pl.BlockSpec((1, 2, 8, P), lambda i: (i, 0, 0, 0)),
            pl.BlockSpec((1, 2, P, 8), lambda i: (i, 0, 0, 0)),
            pl.BlockSpec((1, 2, 1, P), lambda i: (i, 0, 0, 0)),
        ],
        out_specs=pl.BlockSpec((8, P), lambda i: (0, 0)),
        out_shape=jax.ShapeDtypeStruct((8, P), jnp.float32),
        compiler_params=pltpu.CompilerParams(
            dimension_semantics=("arbitrary",),
        ),
        interpret=INTERPRET,
    )(coords.reshape(n_pairs, 2, 8, P), coords_t.reshape(n_pairs, 2, P, 8), t)

    npts = jnp.float32(B * N)
    normal_loss = jnp.sum(res[0]) / npts * jnp.float32(1.0)
    surf_loss = jnp.sum(res[1]) / npts * jnp.float32(1.0)
    return jnp.stack([normal_loss, surf_loss])


# speedup vs baseline: 1.0025x; 1.0025x over previous
import functools

import jax
import jax.numpy as jnp
from jax import lax
from jax.experimental import pallas as pl
from jax.experimental.pallas import tpu as pltpu
from jax.experimental.pallas import tpu_sc as plsc

N_PATCH = 8
K_NN = 16
NEWTON_ITERS = 18
INTERPRET = False
N_CHUNK = 2

_NC = 2
_NS = 16
_NW = _NC * _NS
_RB = 16


def _d2_body(coords_ref, coords_t_ref, out_ref):
    co = coords_ref[0]
    ct = coords_t_ref[0]
    x = co[0:1, :]
    y = co[1:2, :]
    z = co[2:3, :]
    xc = ct[:, 0:1]
    yc = ct[:, 1:2]
    zc = ct[:, 2:3]
    dx = xc - x
    dy = yc - y
    dz = zc - z
    out_ref[0] = dx * dx + dy * dy + dz * dz


def _sort16(a):
    return plsc.sort_key_val(a, a)[0]


def _merge16(a, b):
    return _sort16(jnp.minimum(a, lax.rev(b, (0,))))


def _row_kth(buf, base):
    lvl = [_sort16(buf[pl.ds(base + 16 * c, 16)]) for c in range(64)]
    while len(lvl) > 1:
        lvl = [_merge16(lvl[2 * i], lvl[2 * i + 1]) for i in range(len(lvl) // 2)]
    return jnp.max(lvl[0])


def _sc_select_body(rows_per_w, d2_hbm, t_hbm, buf0, buf1, tv0, tv1, sem0, sem1):
    wid = lax.axis_index("s") * _NC + lax.axis_index("c")
    row0 = wid * rows_per_w
    n_blocks = rows_per_w // _RB
    lane = lax.iota(jnp.int32, 16)

    def dma(g, buf, sem):
        return pltpu.make_async_copy(
            d2_hbm.at[pl.ds((row0 + g * _RB) * 1024, _RB * 1024)], buf, sem
        )

    dma(0, buf0, sem0).start()
    dma(1, buf1, sem1).start()

    def do_block(g, buf, tv, sem):
        dma(g, buf, sem).wait()

        def row_fn(i, tvec):
            t = _row_kth(buf, i * 1024)
            return jnp.where(lane == i, t, tvec)

        tvec = lax.fori_loop(0, _RB, row_fn, jnp.zeros((16,), jnp.float32))

        @pl.when(g + 2 < n_blocks)
        def _():
            dma(g + 2, buf, sem).start()

        tv[...] = tvec
        pltpu.sync_copy(tv, t_hbm.at[pl.ds(row0 + g * _RB, _RB)])

    def super_body(s, carry):
        do_block(s * 2, buf0, tv0, sem0)
        do_block(s * 2 + 1, buf1, tv1, sem1)
        return carry

    lax.fori_loop(0, n_blocks // 2, super_body, jnp.int32(0))


def _cloud_props(co, ct, t):
    x = co[0:1, :]
    y = co[1:2, :]
    z = co[2:3, :]
    xc = ct[:, 0:1]
    yc = ct[:, 1:2]
    zc = ct[:, 2:3]

    dx = xc - x
    dy = yc - y
    dz = zc - z
    d2 = dx * dx + dy * dy + dz * dz

    maskf = jnp.where(d2 <= t, jnp.float32(1.0), jnp.float32(0.0))

    one = jnp.ones_like(x)
    feats = jnp.concatenate(
        [x, y, z, x * x, y * y, z * z, x * y, x * z, y * z, one], axis=0
    )
    S = jnp.dot(feats, maskf, preferred_element_type=jnp.float32)

    Sx = S[0:1]
    Sy = S[1:2]
    Sz = S[2:3]
    cn = S[9:10]
    cxx = S[3:4] - 2.0 * x * Sx + cn * x * x
    cyy = S[4:5] - 2.0 * y * Sy + cn * y * y
    czz = S[5:6] - 2.0 * z * Sz + cn * z * z
    cxy = S[6:7] - x * Sy - y * Sx + cn * x * y
    cxz = S[7:8] - x * Sz - z * Sx + cn * x * z
    cyz = S[8:9] - y * Sz - z * Sy + cn * y * z

    c2 = cxx + cyy + czz
    c1 = (cxx * cyy - cxy * cxy) + (cxx * czz - cxz * cxz) + (cyy * czz - cyz * cyz)
    c0 = (
        cxx * (cyy * czz - cyz * cyz)
        - cxy * (cxy * czz - cyz * cxz)
        + cxz * (cxy * cyz - cyy * cxz)
    )
    lam = -0.01 * c2 - jnp.float32(1e-12)
    for _ in range(NEWTON_ITERS):
        fv = ((c2 - lam) * lam - c1) * lam + c0
        fp = (2.0 * c2 - 3.0 * lam) * lam - c1
        fp = jnp.minimum(fp, jnp.float32(-1e-30))
        lam = lam - fv / fp

    m00 = cxx - lam
    m11 = cyy - lam
    m22 = czz - lam
    a01x = cxy * cyz - cxz * m11
    a01y = cxz * cxy - m00 * cyz
    a01z = m00 * m11 - cxy * cxy
    a02x = cxy * m22 - cxz * cyz
    a02y = cxz * cxz - m00 * m22
    a02z = m00 * cyz - cxy * cxz
    a12x = m11 * m22 - cyz * cyz
    a12y = cyz * cxz - cxy * m22
    a12z = cxy * cyz - m11 * cxz
    n01 = a01x * a01x + a01y * a01y + a01z * a01z
    n02 = a02x * a02x + a02y * a02y + a02z * a02z
    n12 = a12x * a12x + a12y * a12y + a12z * a12z

    use02 = n02 > n01
    vx = jnp.where(use02, a02x, a01x)
    vy = jnp.where(use02, a02y, a01y)
    vz = jnp.where(use02, a02z, a01z)
    nb = jnp.maximum(n01, n02)
    use12 = n12 > nb
    vx = jnp.where(use12, a12x, vx)
    vy = jnp.where(use12, a12y, vy)
    vz = jnp.where(use12, a12z, vz)
    nb = jnp.maximum(nb, n12)

    inv = lax.rsqrt(nb + jnp.float32(1e-38))
    anx = jnp.abs(vx) * inv
    any_ = jnp.abs(vy) * inv
    anz = jnp.abs(vz) * inv
    sv = lam / jnp.maximum(c2, jnp.float32(1e-38))
    return anx, any_, anz, sv


def _pair_body(coords_ref, coords_t_ref, t_ref, out_ref):
    step = pl.program_id(0)
    sx, sy, sz, ssv = _cloud_props(
        coords_ref[0, 0], coords_t_ref[0, 0], t_ref[0, 0]
    )
    dx_, dy_, dz_, dsv = _cloud_props(
        coords_ref[0, 1], coords_t_ref[0, 1], t_ref[0, 1]
    )

    nl = jnp.sqrt((sx - dx_) ** 2 + (sy - dy_) ** 2 + (sz - dz_) ** 2)
    svl = jnp.abs(ssv - dsv)
    zeros = jnp.zeros_like(nl)
    acc = jnp.concatenate([nl, svl, zeros, zeros, zeros, zeros, zeros, zeros], axis=0)

    @pl.when(step == 0)
    def _():
        out_ref[...] = jnp.zeros_like(out_ref)

    out_ref[...] += acc


def _chunk_losses(coords, coords_t, P):
    n_cp = coords.shape[0]
    n_pairs = n_cp // 2
    rows_per_w = n_cp * P // _NW

    d2 = pl.pallas_call(
        _d2_body,
        grid=(n_cp,),
        in_specs=[
            pl.BlockSpec((1, 8, P), lambda i: (i, 0, 0)),
            pl.BlockSpec((1, P, 8), lambda i: (i, 0, 0)),
        ],
        out_specs=pl.BlockSpec((1, P, P), lambda i: (i, 0, 0)),
        out_shape=jax.ShapeDtypeStruct((n_cp, P, P), jnp.float32),
        compiler_params=pltpu.CompilerParams(
            dimension_semantics=("arbitrary",),
        ),
        interpret=INTERPRET,
    )(coords, coords_t)

    mesh = plsc.VectorSubcoreMesh(core_axis_name="c", subcore_axis_name="s")
    sel = functools.partial(
        pl.kernel,
        mesh=mesh,
        out_type=jax.ShapeDtypeStruct((n_cp * P,), jnp.float32),
        compiler_params=pltpu.CompilerParams(needs_layout_passes=False),
        scratch_types=[
            pltpu.VMEM((_RB * 1024,), jnp.float32),
            pltpu.VMEM((_RB * 1024,), jnp.float32),
            pltpu.VMEM((16,), jnp.float32),
            pltpu.VMEM((16,), jnp.float32),
            pltpu.SemaphoreType.DMA,
            pltpu.SemaphoreType.DMA,
        ],
    )(functools.partial(_sc_select_body, rows_per_w))
    t_flat = sel(d2.reshape(n_cp * P * P))

    t = t_flat.reshape(n_pairs, 2, 1, P)

    return pl.pallas_call(
        _pair_body,
        grid=(n_pairs,),
        in_specs=[
            pl.BlockSpec((1, 2, 8, P), lambda i: (i, 0, 0, 0)),
            pl.BlockSpec((1, 2, P, 8), lambda i: (i, 0, 0, 0)),
            pl.BlockSpec((1, 2, 1, P), lambda i: (i, 0, 0, 0)),
        ],
        out_specs=pl.BlockSpec((8, P), lambda i: (0, 0)),
        out_shape=jax.ShapeDtypeStruct((8, P), jnp.float32),
        compiler_params=pltpu.CompilerParams(
            dimension_semantics=("arbitrary",),
        ),
        interpret=INTERPRET,
    )(
        coords.reshape(n_pairs, 2, 8, P),
        coords_t.reshape(n_pairs, 2, P, 8),
        t,
    )


def kernel(srcPC, dstPC):
    B, N, _ = srcPC.shape
    n_pairs = B * N_PATCH
    n_cp = 2 * n_pairs
    P = N // N_PATCH

    s = srcPC.reshape(n_pairs, P, 3)
    d = dstPC.reshape(n_pairs, P, 3)
    pts = jnp.stack([s, d], axis=1)
    pts = jnp.pad(pts, ((0, 0), (0, 0), (0, 0), (0, 5)))
    coords_t = pts.reshape(n_cp, P, 8)
    coords = jnp.swapaxes(coords_t, 1, 2)

    cp_per_chunk = n_cp // N_CHUNK
    res = None
    for ci in range(N_CHUNK):
        lo = ci * cp_per_chunk
        acc = _chunk_losses(
            coords[lo:lo + cp_per_chunk], coords_t[lo:lo + cp_per_chunk], P
        )
        res = acc if res is None else res + acc

    npts = jnp.float32(B * N)
    normal_loss = jnp.sum(res[0]) / npts * jnp.float32(1.0)
    surf_loss = jnp.sum(res[1]) / npts * jnp.float32(1.0)
    return jnp.stack([normal_loss, surf_loss])


# speedup vs baseline: 1.5245x; 1.5207x over previous
import functools

import jax
import jax.numpy as jnp
from jax import lax
from jax.experimental import pallas as pl
from jax.experimental.pallas import tpu as pltpu
from jax.experimental.pallas import tpu_sc as plsc

N_PATCH = 8
K_NN = 16
NEWTON_ITERS = 18
INTERPRET = False
N_CHUNK = 2

_NC = 2
_NS = 16
_NW = _NC * _NS
_RB = 16


def _d2_body(coords_ref, coords_t_ref, out_ref):
    co = coords_ref[0]
    ct = coords_t_ref[0]
    x = co[0:1, :]
    y = co[1:2, :]
    z = co[2:3, :]
    xc = ct[:, 0:1]
    yc = ct[:, 1:2]
    zc = ct[:, 2:3]
    dx = xc - x
    dy = yc - y
    dz = zc - z
    d2v = dx * dx + dy * dy + dz * dz
    for i in range(8):
        out_ref[0, i] = d2v[:, 128 * i:128 * (i + 1)]


def _sort16(a):
    return plsc.sort_key_val(a, a)[0]


def _merge16(a, b):
    return _sort16(jnp.minimum(a, lax.rev(b, (0,))))


def _row_kth(buf, base):
    lvl = [
        _sort16(buf[pl.ds(base + (c // 8) * 2048 + (c % 8) * 16, 16)])
        for c in range(64)
    ]
    while len(lvl) > 1:
        lvl = [_merge16(lvl[2 * i], lvl[2 * i + 1]) for i in range(len(lvl) // 2)]
    return jnp.max(lvl[0])


def _sc_select_body(rows_per_w, d2_hbm, t_hbm, buf0, buf1, tv0, tv1, sem0, sem1):
    wid = lax.axis_index("s") * _NC + lax.axis_index("c")
    row0 = wid * rows_per_w
    n_blocks = rows_per_w // _RB
    lane = lax.iota(jnp.int32, 16)

    def dmas(g, buf, sem):
        row = row0 + g * _RB
        cp = row // 1024
        r = row % 1024
        base = cp * (8 * 1024 * 128) + r * 128
        return [
            pltpu.make_async_copy(
                d2_hbm.at[pl.ds(base + ch * (1024 * 128), _RB * 128)],
                buf.at[pl.ds(ch * (_RB * 128), _RB * 128)],
                sem,
            )
            for ch in range(8)
        ]

    def start(g, buf, sem):
        for c in dmas(g, buf, sem):
            c.start()

    def drain(g, buf, sem):
        for c in dmas(g, buf, sem):
            c.wait()

    start(0, buf0, sem0)
    start(1, buf1, sem1)

    def do_block(g, buf, tv, sem):
        drain(g, buf, sem)

        def row_fn(i, tvec):
            t = _row_kth(buf, i * 128)
            return jnp.where(lane == i, t, tvec)

        tvec = lax.fori_loop(0, _RB, row_fn, jnp.zeros((16,), jnp.float32))

        @pl.when(g + 2 < n_blocks)
        def _():
            start(g + 2, buf, sem)

        tv[...] = tvec
        pltpu.sync_copy(tv, t_hbm.at[pl.ds(row0 + g * _RB, _RB)])

    def super_body(s, carry):
        do_block(s * 2, buf0, tv0, sem0)
        do_block(s * 2 + 1, buf1, tv1, sem1)
        return carry

    lax.fori_loop(0, n_blocks // 2, super_body, jnp.int32(0))


def _cloud_props(co, ct, t):
    x = co[0:1, :]
    y = co[1:2, :]
    z = co[2:3, :]
    xc = ct[:, 0:1]
    yc = ct[:, 1:2]
    zc = ct[:, 2:3]

    dx = xc - x
    dy = yc - y
    dz = zc - z
    d2 = dx * dx + dy * dy + dz * dz

    maskf = jnp.where(d2 <= t, jnp.float32(1.0), jnp.float32(0.0))

    one = jnp.ones_like(x)
    feats = jnp.concatenate(
        [x, y, z, x * x, y * y, z * z, x * y, x * z, y * z, one], axis=0
    )
    S = jnp.dot(feats, maskf, preferred_element_type=jnp.float32)

    Sx = S[0:1]
    Sy = S[1:2]
    Sz = S[2:3]
    cn = S[9:10]
    cxx = S[3:4] - 2.0 * x * Sx + cn * x * x
    cyy = S[4:5] - 2.0 * y * Sy + cn * y * y
    czz = S[5:6] - 2.0 * z * Sz + cn * z * z
    cxy = S[6:7] - x * Sy - y * Sx + cn * x * y
    cxz = S[7:8] - x * Sz - z * Sx + cn * x * z
    cyz = S[8:9] - y * Sz - z * Sy + cn * y * z

    c2 = cxx + cyy + czz
    c1 = (cxx * cyy - cxy * cxy) + (cxx * czz - cxz * cxz) + (cyy * czz - cyz * cyz)
    c0 = (
        cxx * (cyy * czz - cyz * cyz)
        - cxy * (cxy * czz - cyz * cxz)
        + cxz * (cxy * cyz - cyy * cxz)
    )
    lam = -0.01 * c2 - jnp.float32(1e-12)
    for _ in range(NEWTON_ITERS):
        fv = ((c2 - lam) * lam - c1) * lam + c0
        fp = (2.0 * c2 - 3.0 * lam) * lam - c1
        fp = jnp.minimum(fp, jnp.float32(-1e-30))
        lam = lam - fv / fp

    m00 = cxx - lam
    m11 = cyy - lam
    m22 = czz - lam
    a01x = cxy * cyz - cxz * m11
    a01y = cxz * cxy - m00 * cyz
    a01z = m00 * m11 - cxy * cxy
    a02x = cxy * m22 - cxz * cyz
    a02y = cxz * cxz - m00 * m22
    a02z = m00 * cyz - cxy * cxz
    a12x = m11 * m22 - cyz * cyz
    a12y = cyz * cxz - cxy * m22
    a12z = cxy * cyz - m11 * cxz
    n01 = a01x * a01x + a01y * a01y + a01z * a01z
    n02 = a02x * a02x + a02y * a02y + a02z * a02z
    n12 = a12x * a12x + a12y * a12y + a12z * a12z

    use02 = n02 > n01
    vx = jnp.where(use02, a02x, a01x)
    vy = jnp.where(use02, a02y, a01y)
    vz = jnp.where(use02, a02z, a01z)
    nb = jnp.maximum(n01, n02)
    use12 = n12 > nb
    vx = jnp.where(use12, a12x, vx)
    vy = jnp.where(use12, a12y, vy)
    vz = jnp.where(use12, a12z, vz)
    nb = jnp.maximum(nb, n12)

    inv = lax.rsqrt(nb + jnp.float32(1e-38))
    anx = jnp.abs(vx) * inv
    any_ = jnp.abs(vy) * inv
    anz = jnp.abs(vz) * inv
    sv = lam / jnp.maximum(c2, jnp.float32(1e-38))
    return anx, any_, anz, sv


def _pair_body(coords_ref, coords_t_ref, t_ref, out_ref):
    step = pl.program_id(0)
    sx, sy, sz, ssv = _cloud_props(
        coords_ref[0, 0], coords_t_ref[0, 0], t_ref[0, 0]
    )
    dx_, dy_, dz_, dsv = _cloud_props(
        coords_ref[0, 1], coords_t_ref[0, 1], t_ref[0, 1]
    )

    nl = jnp.sqrt((sx - dx_) ** 2 + (sy - dy_) ** 2 + (sz - dz_) ** 2)
    svl = jnp.abs(ssv - dsv)
    zeros = jnp.zeros_like(nl)
    acc = jnp.concatenate([nl, svl, zeros, zeros, zeros, zeros, zeros, zeros], axis=0)

    @pl.when(step == 0)
    def _():
        out_ref[...] = jnp.zeros_like(out_ref)

    out_ref[...] += acc


def _chunk_losses(coords, coords_t, P):
    n_cp = coords.shape[0]
    n_pairs = n_cp // 2
    rows_per_w = n_cp * P // _NW

    d2 = pl.pallas_call(
        _d2_body,
        grid=(n_cp,),
        in_specs=[
            pl.BlockSpec((1, 8, P), lambda i: (i, 0, 0)),
            pl.BlockSpec((1, P, 8), lambda i: (i, 0, 0)),
        ],
        out_specs=pl.BlockSpec((1, 8, P, 128), lambda i: (i, 0, 0, 0)),
        out_shape=jax.ShapeDtypeStruct((n_cp, 8, P, 128), jnp.float32),
        compiler_params=pltpu.CompilerParams(
            dimension_semantics=("arbitrary",),
        ),
        interpret=INTERPRET,
    )(coords, coords_t)

    mesh = plsc.VectorSubcoreMesh(core_axis_name="c", subcore_axis_name="s")
    sel = functools.partial(
        pl.kernel,
        mesh=mesh,
        out_type=jax.ShapeDtypeStruct((n_cp * P,), jnp.float32),
        compiler_params=pltpu.CompilerParams(needs_layout_passes=False),
        scratch_types=[
            pltpu.VMEM((_RB * 1024,), jnp.float32),
            pltpu.VMEM((_RB * 1024,), jnp.float32),
            pltpu.VMEM((16,), jnp.float32),
            pltpu.VMEM((16,), jnp.float32),
            pltpu.SemaphoreType.DMA,
            pltpu.SemaphoreType.DMA,
        ],
    )(functools.partial(_sc_select_body, rows_per_w))
    t_flat = sel(d2.reshape(n_cp * P * P))

    t = t_flat.reshape(n_pairs, 2, 1, P)

    return pl.pallas_call(
        _pair_body,
        grid=(n_pairs,),
        in_specs=[
            pl.BlockSpec((1, 2, 8, P), lambda i: (i, 0, 0, 0)),
            pl.BlockSpec((1, 2, P, 8), lambda i: (i, 0, 0, 0)),
            pl.BlockSpec((1, 2, 1, P), lambda i: (i, 0, 0, 0)),
        ],
        out_specs=pl.BlockSpec((8, P), lambda i: (0, 0)),
        out_shape=jax.ShapeDtypeStruct((8, P), jnp.float32),
        compiler_params=pltpu.CompilerParams(
            dimension_semantics=("arbitrary",),
        ),
        interpret=INTERPRET,
    )(
        coords.reshape(n_pairs, 2, 8, P),
        coords_t.reshape(n_pairs, 2, P, 8),
        t,
    )


def kernel(srcPC, dstPC):
    B, N, _ = srcPC.shape
    n_pairs = B * N_PATCH
    n_cp = 2 * n_pairs
    P = N // N_PATCH

    s = srcPC.reshape(n_pairs, P, 3)
    d = dstPC.reshape(n_pairs, P, 3)
    pts = jnp.stack([s, d], axis=1)
    pts = jnp.pad(pts, ((0, 0), (0, 0), (0, 0), (0, 5)))
    coords_t = pts.reshape(n_cp, P, 8)
    coords = jnp.swapaxes(coords_t, 1, 2)

    cp_per_chunk = n_cp // N_CHUNK
    res = None
    for ci in range(N_CHUNK):
        lo = ci * cp_per_chunk
        acc = _chunk_losses(
            coords[lo:lo + cp_per_chunk], coords_t[lo:lo + cp_per_chunk], P
        )
        res = acc if res is None else res + acc

    npts = jnp.float32(B * N)
    normal_loss = jnp.sum(res[0]) / npts * jnp.float32(1.0)
    surf_loss = jnp.sum(res[1]) / npts * jnp.float32(1.0)
    return jnp.stack([normal_loss, surf_loss])


# speedup vs baseline: 1.7157x; 1.1254x over previous
import functools

import jax
import jax.numpy as jnp
from jax import lax
from jax.experimental import pallas as pl
from jax.experimental.pallas import tpu as pltpu
from jax.experimental.pallas import tpu_sc as plsc

N_PATCH = 8
K_NN = 16
NEWTON_ITERS = 18
INTERPRET = False
N_CHUNK = 4

_NC = 2
_NS = 16
_NW = _NC * _NS
_RB = 16


def _d2_of(co, ct):
    x = co[0:1, :]
    y = co[1:2, :]
    z = co[2:3, :]
    xc = ct[:, 0:1]
    yc = ct[:, 1:2]
    zc = ct[:, 2:3]
    dx = xc - x
    dy = yc - y
    dz = zc - z
    return dx * dx + dy * dy + dz * dz


def _d2_body(sc_ref, st_ref, dc_ref, dt_ref, out_ref):
    d2s = _d2_of(sc_ref[0], st_ref[0])
    d2d = _d2_of(dc_ref[0], dt_ref[0])
    for i in range(8):
        out_ref[0, 0, i] = d2s[:, 128 * i:128 * (i + 1)]
        out_ref[0, 1, i] = d2d[:, 128 * i:128 * (i + 1)]


def _sort16(a):
    return plsc.sort_key_val(a, a)[0]


def _merge16(a, b):
    return _sort16(jnp.minimum(a, lax.rev(b, (0,))))


def _row_kth(buf, base):
    lvl = [
        _sort16(buf[pl.ds(base + (c // 8) * (_RB * 128) + (c % 8) * 16, 16)])
        for c in range(64)
    ]
    while len(lvl) > 1:
        lvl = [_merge16(lvl[2 * i], lvl[2 * i + 1]) for i in range(len(lvl) // 2)]
    return jnp.max(lvl[0])


def _sc_select_body(rows_per_w, d2_hbm, t_hbm, buf0, buf1, tv0, tv1, sem0, sem1):
    wid = lax.axis_index("s") * _NC + lax.axis_index("c")
    row0 = wid * rows_per_w
    n_blocks = rows_per_w // _RB
    lane = lax.iota(jnp.int32, 16)

    def dmas(g, buf, sem):
        row = row0 + g * _RB
        cp = row // 1024
        r = row % 1024
        base = cp * (8 * 1024 * 128) + r * 128
        return [
            pltpu.make_async_copy(
                d2_hbm.at[pl.ds(base + ch * (1024 * 128), _RB * 128)],
                buf.at[pl.ds(ch * (_RB * 128), _RB * 128)],
                sem,
            )
            for ch in range(8)
        ]

    def start(g, buf, sem):
        for c in dmas(g, buf, sem):
            c.start()

    def drain(g, buf, sem):
        for c in dmas(g, buf, sem):
            c.wait()

    start(0, buf0, sem0)
    start(1, buf1, sem1)

    def do_block(g, buf, tv, sem):
        drain(g, buf, sem)

        def row_fn(i, tvec):
            t = _row_kth(buf, i * 128)
            return jnp.where(lane == i, t, tvec)

        tvec = lax.fori_loop(0, _RB, row_fn, jnp.zeros((16,), jnp.float32))

        @pl.when(g + 2 < n_blocks)
        def _():
            start(g + 2, buf, sem)

        tv[...] = tvec
        pltpu.sync_copy(tv, t_hbm.at[pl.ds(row0 + g * _RB, _RB)])

    def super_body(s, carry):
        do_block(s * 2, buf0, tv0, sem0)
        do_block(s * 2 + 1, buf1, tv1, sem1)
        return carry

    lax.fori_loop(0, n_blocks // 2, super_body, jnp.int32(0))


def _cloud_props(co, ct, t):
    x = co[0:1, :]
    y = co[1:2, :]
    z = co[2:3, :]
    xc = ct[:, 0:1]
    yc = ct[:, 1:2]
    zc = ct[:, 2:3]

    dx = xc - x
    dy = yc - y
    dz = zc - z
    d2 = dx * dx + dy * dy + dz * dz

    maskf = jnp.where(d2 <= t, jnp.float32(1.0), jnp.float32(0.0))

    one = jnp.ones_like(x)
    feats = jnp.concatenate(
        [x, y, z, x * x, y * y, z * z, x * y, x * z, y * z, one], axis=0
    )
    S = jnp.dot(feats, maskf, preferred_element_type=jnp.float32)

    Sx = S[0:1]
    Sy = S[1:2]
    Sz = S[2:3]
    cn = S[9:10]
    cxx = S[3:4] - 2.0 * x * Sx + cn * x * x
    cyy = S[4:5] - 2.0 * y * Sy + cn * y * y
    czz = S[5:6] - 2.0 * z * Sz + cn * z * z
    cxy = S[6:7] - x * Sy - y * Sx + cn * x * y
    cxz = S[7:8] - x * Sz - z * Sx + cn * x * z
    cyz = S[8:9] - y * Sz - z * Sy + cn * y * z

    c2 = cxx + cyy + czz
    c1 = (cxx * cyy - cxy * cxy) + (cxx * czz - cxz * cxz) + (cyy * czz - cyz * cyz)
    c0 = (
        cxx * (cyy * czz - cyz * cyz)
        - cxy * (cxy * czz - cyz * cxz)
        + cxz * (cxy * cyz - cyy * cxz)
    )
    lam = -0.01 * c2 - jnp.float32(1e-12)
    for _ in range(NEWTON_ITERS):
        fv = ((c2 - lam) * lam - c1) * lam + c0
        fp = (2.0 * c2 - 3.0 * lam) * lam - c1
        fp = jnp.minimum(fp, jnp.float32(-1e-30))
        lam = lam - fv / fp

    m00 = cxx - lam
    m11 = cyy - lam
    m22 = czz - lam
    a01x = cxy * cyz - cxz * m11
    a01y = cxz * cxy - m00 * cyz
    a01z = m00 * m11 - cxy * cxy
    a02x = cxy * m22 - cxz * cyz
    a02y = cxz * cxz - m00 * m22
    a02z = m00 * cyz - cxy * cxz
    a12x = m11 * m22 - cyz * cyz
    a12y = cyz * cxz - cxy * m22
    a12z = cxy * cyz - m11 * cxz
    n01 = a01x * a01x + a01y * a01y + a01z * a01z
    n02 = a02x * a02x + a02y * a02y + a02z * a02z
    n12 = a12x * a12x + a12y * a12y + a12z * a12z

    use02 = n02 > n01
    vx = jnp.where(use02, a02x, a01x)
    vy = jnp.where(use02, a02y, a01y)
    vz = jnp.where(use02, a02z, a01z)
    nb = jnp.maximum(n01, n02)
    use12 = n12 > nb
    vx = jnp.where(use12, a12x, vx)
    vy = jnp.where(use12, a12y, vy)
    vz = jnp.where(use12, a12z, vz)
    nb = jnp.maximum(nb, n12)

    inv = lax.rsqrt(nb + jnp.float32(1e-38))
    anx = jnp.abs(vx) * inv
    any_ = jnp.abs(vy) * inv
    anz = jnp.abs(vz) * inv
    sv = lam / jnp.maximum(c2, jnp.float32(1e-38))
    return anx, any_, anz, sv


def _pair_body(sc_ref, st_ref, dc_ref, dt_ref, t_ref, out_ref):
    step = pl.program_id(0)
    sx, sy, sz, ssv = _cloud_props(sc_ref[0], st_ref[0], t_ref[0, 0])
    dx_, dy_, dz_, dsv = _cloud_props(dc_ref[0], dt_ref[0], t_ref[0, 1])

    nl = jnp.sqrt((sx - dx_) ** 2 + (sy - dy_) ** 2 + (sz - dz_) ** 2)
    svl = jnp.abs(ssv - dsv)
    zeros = jnp.zeros_like(nl)
    acc = jnp.concatenate([nl, svl, zeros, zeros, zeros, zeros, zeros, zeros], axis=0)

    @pl.when(step == 0)
    def _():
        out_ref[...] = jnp.zeros_like(out_ref)

    out_ref[...] += acc


def _chunk_losses(ss, st, ds, dt, P):
    n_pairs = ss.shape[0]
    n_cp = 2 * n_pairs
    rows_per_w = n_cp * P // _NW

    d2 = pl.pallas_call(
        _d2_body,
        grid=(n_pairs,),
        in_specs=[
            pl.BlockSpec((1, 3, P), lambda i: (i, 0, 0)),
            pl.BlockSpec((1, P, 3), lambda i: (i, 0, 0)),
            pl.BlockSpec((1, 3, P), lambda i: (i, 0, 0)),
            pl.BlockSpec((1, P, 3), lambda i: (i, 0, 0)),
        ],
        out_specs=pl.BlockSpec((1, 2, 8, P, 128), lambda i: (i, 0, 0, 0, 0)),
        out_shape=jax.ShapeDtypeStruct((n_pairs, 2, 8, P, 128), jnp.float32),
        compiler_params=pltpu.CompilerParams(
            dimension_semantics=("arbitrary",),
        ),
        interpret=INTERPRET,
    )(st, ss, dt, ds)

    mesh = plsc.VectorSubcoreMesh(core_axis_name="c", subcore_axis_name="s")
    sel = functools.partial(
        pl.kernel,
        mesh=mesh,
        out_type=jax.ShapeDtypeStruct((n_cp * P,), jnp.float32),
        compiler_params=pltpu.CompilerParams(needs_layout_passes=False),
        scratch_types=[
            pltpu.VMEM((_RB * 8 * 128,), jnp.float32),
            pltpu.VMEM((_RB * 8 * 128,), jnp.float32),
            pltpu.VMEM((16,), jnp.float32),
            pltpu.VMEM((16,), jnp.float32),
            pltpu.SemaphoreType.DMA,
            pltpu.SemaphoreType.DMA,
        ],
    )(functools.partial(_sc_select_body, rows_per_w))
    t_flat = sel(d2.reshape(n_cp * P * P))

    t = t_flat.reshape(n_pairs, 2, 1, P)

    return pl.pallas_call(
        _pair_body,
        grid=(n_pairs,),
        in_specs=[
            pl.BlockSpec((1, 3, P), lambda i: (i, 0, 0)),
            pl.BlockSpec((1, P, 3), lambda i: (i, 0, 0)),
            pl.BlockSpec((1, 3, P), lambda i: (i, 0, 0)),
            pl.BlockSpec((1, P, 3), lambda i: (i, 0, 0)),
            pl.BlockSpec((1, 2, 1, P), lambda i: (i, 0, 0, 0)),
        ],
        out_specs=pl.BlockSpec((8, P), lambda i: (0, 0)),
        out_shape=jax.ShapeDtypeStruct((8, P), jnp.float32),
        compiler_params=pltpu.CompilerParams(
            dimension_semantics=("arbitrary",),
        ),
        interpret=INTERPRET,
    )(st, ss, dt, ds, t)


def kernel(srcPC, dstPC):
    B, N, _ = srcPC.shape
    n_pairs = B * N_PATCH
    P = N // N_PATCH

    s_all = srcPC.reshape(n_pairs, P, 3)
    d_all = dstPC.reshape(n_pairs, P, 3)

    pairs_per_chunk = n_pairs // N_CHUNK
    res = None
    for ci in range(N_CHUNK):
        lo = ci * pairs_per_chunk
        hi = lo + pairs_per_chunk
        ss = s_all[lo:hi]
        ds = d_all[lo:hi]
        st = jnp.swapaxes(ss, 1, 2)
        dt = jnp.swapaxes(ds, 1, 2)
        acc = _chunk_losses(ss, st, ds, dt, P)
        res = acc if res is None else res + acc

    npts = jnp.float32(B * N)
    normal_loss = jnp.sum(res[0]) / npts * jnp.float32(1.0)
    surf_loss = jnp.sum(res[1]) / npts * jnp.float32(1.0)
    return jnp.stack([normal_loss, surf_loss])


# speedup vs baseline: 1.7709x; 1.0322x over previous
import functools

import jax
import jax.numpy as jnp
from jax import lax
from jax.experimental import pallas as pl
from jax.experimental.pallas import tpu as pltpu
from jax.experimental.pallas import tpu_sc as plsc

N_PATCH = 8
K_NN = 16
NEWTON_ITERS = 18
INTERPRET = False
N_CHUNK = 4

_NC = 2
_NS = 16
_NW = _NC * _NS
_RB = 16


def _d2_of(co, ct):
    x = co[0:1, :]
    y = co[1:2, :]
    z = co[2:3, :]
    xc = ct[:, 0:1]
    yc = ct[:, 1:2]
    zc = ct[:, 2:3]
    dx = xc - x
    dy = yc - y
    dz = zc - z
    return dx * dx + dy * dy + dz * dz


def _d2_body(sc_ref, st_ref, dc_ref, dt_ref, out_ref):
    d2s = _d2_of(sc_ref[0], st_ref[0])
    d2d = _d2_of(dc_ref[0], dt_ref[0])
    for i in range(8):
        out_ref[0, 0, i] = d2s[:, 128 * i:128 * (i + 1)]
        out_ref[0, 1, i] = d2d[:, 128 * i:128 * (i + 1)]


def _sort16(a):
    return plsc.sort_key_val(a, a)[0]


def _merge16(a, b):
    return _sort16(jnp.minimum(a, lax.rev(b, (0,))))


def _row_kth(buf, base):
    lvl = [
        _sort16(buf[pl.ds(base + (c // 8) * (_RB * 128) + (c % 8) * 16, 16)])
        for c in range(64)
    ]
    while len(lvl) > 1:
        lvl = [_merge16(lvl[2 * i], lvl[2 * i + 1]) for i in range(len(lvl) // 2)]
    return jnp.max(lvl[0])


def _sc_select_body(rows_per_w, d2_hbm, t_hbm, buf0, buf1, tv0, tv1, sem0, sem1):
    wid = lax.axis_index("s") * _NC + lax.axis_index("c")
    row0 = wid * rows_per_w
    n_blocks = rows_per_w // _RB
    lane = lax.iota(jnp.int32, 16)

    def dmas(g, buf, sem):
        row = row0 + g * _RB
        cp = row // 1024
        r = row % 1024
        base = cp * (8 * 1024 * 128) + r * 128
        return [
            pltpu.make_async_copy(
                d2_hbm.at[pl.ds(base + ch * (1024 * 128), _RB * 128)],
                buf.at[pl.ds(ch * (_RB * 128), _RB * 128)],
                sem,
            )
            for ch in range(8)
        ]

    def start(g, buf, sem):
        for c in dmas(g, buf, sem):
            c.start()

    def drain(g, buf, sem):
        for c in dmas(g, buf, sem):
            c.wait()

    start(0, buf0, sem0)
    start(1, buf1, sem1)

    def do_block(g, buf, tv, sem):
        drain(g, buf, sem)

        def row_fn(i, tvec):
            t = _row_kth(buf, i * 128)
            return jnp.where(lane == i, t, tvec)

        tvec = lax.fori_loop(0, _RB, row_fn, jnp.zeros((16,), jnp.float32))

        @pl.when(g + 2 < n_blocks)
        def _():
            start(g + 2, buf, sem)

        tv[...] = tvec
        pltpu.sync_copy(tv, t_hbm.at[pl.ds(row0 + g * _RB, _RB)])

    def super_body(s, carry):
        do_block(s * 2, buf0, tv0, sem0)
        do_block(s * 2 + 1, buf1, tv1, sem1)
        return carry

    lax.fori_loop(0, n_blocks // 2, super_body, jnp.int32(0))


def _cloud_props(co, ct, t):
    x = co[0:1, :]
    y = co[1:2, :]
    z = co[2:3, :]
    xc = ct[:, 0:1]
    yc = ct[:, 1:2]
    zc = ct[:, 2:3]

    dx = xc - x
    dy = yc - y
    dz = zc - z
    d2 = dx * dx + dy * dy + dz * dz

    maskf = jnp.where(d2 <= t, jnp.float32(1.0), jnp.float32(0.0))

    one = jnp.ones_like(x)
    feats = jnp.concatenate(
        [x, y, z, x * x, y * y, z * z, x * y, x * z, y * z, one], axis=0
    )
    S = jnp.dot(feats, maskf, preferred_element_type=jnp.float32)

    Sx = S[0:1]
    Sy = S[1:2]
    Sz = S[2:3]
    cn = S[9:10]
    cxx = S[3:4] - 2.0 * x * Sx + cn * x * x
    cyy = S[4:5] - 2.0 * y * Sy + cn * y * y
    czz = S[5:6] - 2.0 * z * Sz + cn * z * z
    cxy = S[6:7] - x * Sy - y * Sx + cn * x * y
    cxz = S[7:8] - x * Sz - z * Sx + cn * x * z
    cyz = S[8:9] - y * Sz - z * Sy + cn * y * z

    c2 = cxx + cyy + czz
    c1 = (cxx * cyy - cxy * cxy) + (cxx * czz - cxz * cxz) + (cyy * czz - cyz * cyz)
    c0 = (
        cxx * (cyy * czz - cyz * cyz)
        - cxy * (cxy * czz - cyz * cxz)
        + cxz * (cxy * cyz - cyy * cxz)
    )
    lam = -0.01 * c2 - jnp.float32(1e-12)
    for _ in range(NEWTON_ITERS):
        fv = ((c2 - lam) * lam - c1) * lam + c0
        fp = (2.0 * c2 - 3.0 * lam) * lam - c1
        fp = jnp.minimum(fp, jnp.float32(-1e-30))
        lam = lam - fv / fp

    m00 = cxx - lam
    m11 = cyy - lam
    m22 = czz - lam
    a01x = cxy * cyz - cxz * m11
    a01y = cxz * cxy - m00 * cyz
    a01z = m00 * m11 - cxy * cxy
    a02x = cxy * m22 - cxz * cyz
    a02y = cxz * cxz - m00 * m22
    a02z = m00 * cyz - cxy * cxz
    a12x = m11 * m22 - cyz * cyz
    a12y = cyz * cxz - cxy * m22
    a12z = cxy * cyz - m11 * cxz
    n01 = a01x * a01x + a01y * a01y + a01z * a01z
    n02 = a02x * a02x + a02y * a02y + a02z * a02z
    n12 = a12x * a12x + a12y * a12y + a12z * a12z

    use02 = n02 > n01
    vx = jnp.where(use02, a02x, a01x)
    vy = jnp.where(use02, a02y, a01y)
    vz = jnp.where(use02, a02z, a01z)
    nb = jnp.maximum(n01, n02)
    use12 = n12 > nb
    vx = jnp.where(use12, a12x, vx)
    vy = jnp.where(use12, a12y, vy)
    vz = jnp.where(use12, a12z, vz)
    nb = jnp.maximum(nb, n12)

    inv = lax.rsqrt(nb + jnp.float32(1e-38))
    anx = jnp.abs(vx) * inv
    any_ = jnp.abs(vy) * inv
    anz = jnp.abs(vz) * inv
    sv = lam / jnp.maximum(c2, jnp.float32(1e-38))
    return anx, any_, anz, sv


def _pair_body(sc_ref, st_ref, dc_ref, dt_ref, t_ref, out_ref):
    step = pl.program_id(0)
    sx, sy, sz, ssv = _cloud_props(sc_ref[0], st_ref[0], t_ref[0, 0])
    dx_, dy_, dz_, dsv = _cloud_props(dc_ref[0], dt_ref[0], t_ref[0, 1])

    nl = jnp.sqrt((sx - dx_) ** 2 + (sy - dy_) ** 2 + (sz - dz_) ** 2)
    svl = jnp.abs(ssv - dsv)
    zeros = jnp.zeros_like(nl)
    acc = jnp.concatenate([nl, svl, zeros, zeros, zeros, zeros, zeros, zeros], axis=0)

    @pl.when(step == 0)
    def _():
        out_ref[...] = jnp.zeros_like(out_ref)

    out_ref[...] += acc


def _chunk_losses(ss, st, ds, dt, lo, n_pairs, P):
    n_cp = 2 * n_pairs
    rows_per_w = n_cp * P // _NW

    d2 = pl.pallas_call(
        _d2_body,
        grid=(n_pairs,),
        in_specs=[
            pl.BlockSpec((1, 3, P), lambda i: (lo + i, 0, 0)),
            pl.BlockSpec((1, P, 3), lambda i: (lo + i, 0, 0)),
            pl.BlockSpec((1, 3, P), lambda i: (lo + i, 0, 0)),
            pl.BlockSpec((1, P, 3), lambda i: (lo + i, 0, 0)),
        ],
        out_specs=pl.BlockSpec((1, 2, 8, P, 128), lambda i: (i, 0, 0, 0, 0)),
        out_shape=jax.ShapeDtypeStruct((n_pairs, 2, 8, P, 128), jnp.float32),
        compiler_params=pltpu.CompilerParams(
            dimension_semantics=("arbitrary",),
        ),
        interpret=INTERPRET,
    )(st, ss, dt, ds)

    mesh = plsc.VectorSubcoreMesh(core_axis_name="c", subcore_axis_name="s")
    sel = functools.partial(
        pl.kernel,
        mesh=mesh,
        out_type=jax.ShapeDtypeStruct((n_cp * P,), jnp.float32),
        compiler_params=pltpu.CompilerParams(needs_layout_passes=False),
        scratch_types=[
            pltpu.VMEM((_RB * 8 * 128,), jnp.float32),
            pltpu.VMEM((_RB * 8 * 128,), jnp.float32),
            pltpu.VMEM((16,), jnp.float32),
            pltpu.VMEM((16,), jnp.float32),
            pltpu.SemaphoreType.DMA,
            pltpu.SemaphoreType.DMA,
        ],
    )(functools.partial(_sc_select_body, rows_per_w))
    t_flat = sel(d2.reshape(n_cp * P * P))

    t = t_flat.reshape(n_pairs, 2, 1, P)

    return pl.pallas_call(
        _pair_body,
        grid=(n_pairs,),
        in_specs=[
            pl.BlockSpec((1, 3, P), lambda i: (lo + i, 0, 0)),
            pl.BlockSpec((1, P, 3), lambda i: (lo + i, 0, 0)),
            pl.BlockSpec((1, 3, P), lambda i: (lo + i, 0, 0)),
            pl.BlockSpec((1, P, 3), lambda i: (lo + i, 0, 0)),
            pl.BlockSpec((1, 2, 1, P), lambda i: (i, 0, 0, 0)),
        ],
        out_specs=pl.BlockSpec((8, P), lambda i: (0, 0)),
        out_shape=jax.ShapeDtypeStruct((8, P), jnp.float32),
        compiler_params=pltpu.CompilerParams(
            dimension_semantics=("arbitrary",),
        ),
        interpret=INTERPRET,
    )(st, ss, dt, ds, t)


def kernel(srcPC, dstPC):
    B, N, _ = srcPC.shape
    n_pairs = B * N_PATCH
    P = N // N_PATCH

    s_all = srcPC.reshape(n_pairs, P, 3)
    d_all = dstPC.reshape(n_pairs, P, 3)
    st_all = jnp.swapaxes(s_all, 1, 2)
    dt_all = jnp.swapaxes(d_all, 1, 2)

    pairs_per_chunk = n_pairs // N_CHUNK
    res = None
    for ci in range(N_CHUNK):
        lo = ci * pairs_per_chunk
        acc = _chunk_losses(s_all, st_all, d_all, dt_all, lo, pairs_per_chunk, P)
        res = acc if res is None else res + acc

    npts = jnp.float32(B * N)
    normal_loss = jnp.sum(res[0]) / npts * jnp.float32(1.0)
    surf_loss = jnp.sum(res[1]) / npts * jnp.float32(1.0)
    return jnp.stack([normal_loss, surf_loss])
